# baseline (speedup 1.0000x reference)
import math

import jax
import jax.numpy as jnp
from jax.experimental import pallas as pl
from jax.experimental.pallas import tpu as pltpu

_HID = 64
_NL = 2
_ALPHA = 0.1
_LAMDA = 1.0
_EPS = 1e-5
_F32 = jnp.float32
_BF16 = jnp.bfloat16
_PREC = jax.lax.Precision.HIGHEST
_K = 4


def _ln_relu(h, g, b):
    mu = jnp.mean(h, axis=1, keepdims=True)
    xc = h - mu
    var = jnp.mean(xc * xc, axis=1, keepdims=True)
    t = xc * jax.lax.rsqrt(var + _EPS) * g + b
    return jnp.maximum(t, 0.0)


def _gcn2(a, t, dinv, w, beta):
    gm = dinv * t
    gm16t = jnp.transpose(gm.astype(_BF16))
    aggt = jax.lax.dot_general(gm16t, a, (((1,), (0,)), ((), ())),
                               preferred_element_type=_F32)
    agg = jnp.transpose(aggt)
    xx = (1.0 - _ALPHA) * (dinv * (agg + gm)) + _ALPHA * t
    return (1.0 - beta) * xx + beta * jax.lax.dot_general(
        xx, w, (((1,), (0,)), ((), ())),
        precision=_PREC, preferred_element_type=_F32)


def _fused_kernel(x_ref, adj_ref, pw_ref, pb_ref, g_ref, b_ref, w0_ref,
                  w1_ref, ow_ref, ob_ref, o_ref, a16_ref, deg_ref):
    beta1 = math.log(_LAMDA / 1.0 + 1.0)
    beta2 = math.log(_LAMDA / 2.0 + 1.0)
    i = pl.program_id(1)
    bc = adj_ref.shape[1]

    chunk = adj_ref[0].astype(_BF16)
    a16_ref[pl.ds(i * bc, bc), :] = chunk
    ones = jnp.ones((1, bc), _BF16)
    part = jax.lax.dot_general(ones, chunk, (((1,), (0,)), ((), ())),
                               preferred_element_type=_F32)

    @pl.when(i == 0)
    def _init():
        deg_ref[...] = part

    @pl.when(i > 0)
    def _acc():
        deg_ref[...] += part

    @pl.when(i == _K - 1)
    def _compute():
        a = a16_ref[...]
        dinv = jnp.transpose(jax.lax.rsqrt(deg_ref[...] + 1.0))
        h = jax.lax.dot_general(x_ref[0], pw_ref[...], (((1,), (1,)), ((), ())),
                                precision=_PREC, preferred_element_type=_F32)
        h = h + pb_ref[...]
        t = _ln_relu(h, g_ref[0:1, :], b_ref[0:1, :])
        h = h + _gcn2(a, t, dinv, w0_ref[...], beta1)
        t = _ln_relu(h, g_ref[1:2, :], b_ref[1:2, :])
        h = h + _gcn2(a, t, dinv, w1_ref[...], beta2)
        o = jnp.sum(h * ow_ref[...], axis=1, keepdims=True)
        o_ref[0] = o + ob_ref[0, 0]


def kernel(x, adj, proj_W, proj_b, ln_g, ln_b, conv_W, pred_W, pred_b):
    bs, N, D = x.shape
    BC = N // _K
    return pl.pallas_call(
        _fused_kernel,
        grid=(bs, _K),
        in_specs=[
            pl.BlockSpec((1, N, D), lambda b, i: (b, 0, 0)),
            pl.BlockSpec((1, BC, N), lambda b, i: (b, i, 0)),
            pl.BlockSpec((_HID, D), lambda b, i: (0, 0)),
            pl.BlockSpec((1, _HID), lambda b, i: (0, 0)),
            pl.BlockSpec((_NL, _HID), lambda b, i: (0, 0)),
            pl.BlockSpec((_NL, _HID), lambda b, i: (0, 0)),
            pl.BlockSpec((_HID, _HID), lambda b, i: (0, 0)),
            pl.BlockSpec((_HID, _HID), lambda b, i: (0, 0)),
            pl.BlockSpec((1, _HID), lambda b, i: (0, 0)),
            pl.BlockSpec((1, 1), lambda b, i: (0, 0)),
        ],
        out_specs=pl.BlockSpec((1, N, 1), lambda b, i: (b, 0, 0)),
        out_shape=jax.ShapeDtypeStruct((bs, N, 1), _F32),
        scratch_shapes=[
            pltpu.VMEM((N, N), _BF16),
            pltpu.VMEM((1, N), _F32),
        ],
    )(x, adj, proj_W, proj_b.reshape(1, _HID), ln_g, ln_b, conv_W[0],
      conv_W[1], pred_W, pred_b.reshape(1, 1))


# speedup vs baseline: 1.1515x; 1.1515x over previous
import math

import jax
import jax.numpy as jnp
from jax.experimental import pallas as pl

_HID = 64
_NL = 2
_ALPHA = 0.1
_LAMDA = 1.0
_EPS = 1e-5
_F32 = jnp.float32
_BF16 = jnp.bfloat16
_PREC = jax.lax.Precision.HIGHEST


def _ln_relu(h, g, b):
    mu = jnp.mean(h, axis=1, keepdims=True)
    xc = h - mu
    var = jnp.mean(xc * xc, axis=1, keepdims=True)
    t = xc * jax.lax.rsqrt(var + _EPS) * g + b
    return jnp.maximum(t, 0.0)


def _gcn2(a, t, dinv, w, beta):
    gm = dinv * t
    gm16t = jnp.transpose(gm.astype(_BF16))
    aggt = jax.lax.dot_general(gm16t, a, (((1,), (0,)), ((), ())),
                               preferred_element_type=_F32)
    agg = jnp.transpose(aggt)
    xx = (1.0 - _ALPHA) * (dinv * (agg + gm)) + _ALPHA * t
    return (1.0 - beta) * xx + beta * jax.lax.dot_general(
        xx, w, (((1,), (0,)), ((), ())),
        precision=_PREC, preferred_element_type=_F32)


def _fused_kernel(x_ref, adj_ref, pw_ref, pb_ref, g_ref, b_ref, w0_ref,
                  w1_ref, ow_ref, ob_ref, o_ref):
    beta1 = math.log(_LAMDA / 1.0 + 1.0)
    beta2 = math.log(_LAMDA / 2.0 + 1.0)
    h = jax.lax.dot_general(x_ref[0], pw_ref[...], (((1,), (1,)), ((), ())),
                            precision=_PREC, preferred_element_type=_F32)
    h = h + pb_ref[...]
    a = adj_ref[0].astype(_BF16)
    ones = jnp.ones((1, a.shape[0]), _BF16)
    deg = jax.lax.dot_general(ones, a, (((1,), (0,)), ((), ())),
                              preferred_element_type=_F32)
    dinv = jnp.transpose(jax.lax.rsqrt(deg + 1.0))
    t = _ln_relu(h, g_ref[0:1, :], b_ref[0:1, :])
    h = h + _gcn2(a, t, dinv, w0_ref[...], beta1)
    t = _ln_relu(h, g_ref[1:2, :], b_ref[1:2, :])
    h = h + _gcn2(a, t, dinv, w1_ref[...], beta2)
    o = jnp.sum(h * ow_ref[...], axis=1, keepdims=True)
    o_ref[0] = o + ob_ref[0, 0]


def kernel(x, adj, proj_W, proj_b, ln_g, ln_b, conv_W, pred_W, pred_b):
    bs, N, D = x.shape
    return pl.pallas_call(
        _fused_kernel,
        grid=(bs,),
        in_specs=[
            pl.BlockSpec((1, N, D), lambda b: (b, 0, 0)),
            pl.BlockSpec((1, N, N), lambda b: (b, 0, 0)),
            pl.BlockSpec((_HID, D), lambda b: (0, 0)),
            pl.BlockSpec((1, _HID), lambda b: (0, 0)),
            pl.BlockSpec((_NL, _HID), lambda b: (0, 0)),
            pl.BlockSpec((_NL, _HID), lambda b: (0, 0)),
            pl.BlockSpec((_HID, _HID), lambda b: (0, 0)),
            pl.BlockSpec((_HID, _HID), lambda b: (0, 0)),
            pl.BlockSpec((1, _HID), lambda b: (0, 0)),
            pl.BlockSpec((1, 1), lambda b: (0, 0)),
        ],
        out_specs=pl.BlockSpec((1, N, 1), lambda b: (b, 0, 0)),
        out_shape=jax.ShapeDtypeStruct((bs, N, 1), _F32),
    )(x, adj, proj_W, proj_b.reshape(1, _HID), ln_g, ln_b, conv_W[0],
      conv_W[1], pred_W, pred_b.reshape(1, 1))
